# baseline (speedup 1.0000x reference)
import functools

import jax
import jax.numpy as jnp
from jax import lax
from jax.experimental import pallas as pl
from jax.experimental.pallas import tpu as pltpu
from jax.experimental.pallas import tpu_sc as plsc

N = 10000
E = 320000
D_IN = 128
D_H = 128
D_OUT = 40
D_OUT_PAD = 64

NPAD = 10240
BLK = 512
CHUNK = 128
NW = 32
NCH = 80
HCH = NCH // 2
EPT = NCH * CHUNK
EPAD = NW * EPT
ZROWS = NPAD // 16
NBUF = 2


def _norm_from_deg(deg_ref):
    deg = deg_ref[0, :, :1] + deg_ref[1, :, :1]
    return jnp.where(deg > 0, lax.rsqrt(jnp.maximum(deg, 1.0)), 0.0)


def _entry_body(x_ref, w_ref, dsrc_ref, o_ref):
    ns = _norm_from_deg(dsrc_ref)
    o_ref[...] = jnp.dot(x_ref[...], w_ref[...],
                         preferred_element_type=jnp.float32) * ns


def _mid_body(p_ref, ddst_ref, dsrc_ref, b_ref, w_ref, o_ref):
    agg = p_ref[0] + p_ref[1]
    nd = _norm_from_deg(ddst_ref)
    h = jnp.maximum(agg * nd + b_ref[...], 0.0)
    ns = _norm_from_deg(dsrc_ref)
    o_ref[...] = jnp.dot(h, w_ref[...],
                         preferred_element_type=jnp.float32) * ns


def _pre_final_body(p_ref, ddst_ref, dsrc_ref, b_ref, o_ref):
    agg = p_ref[0] + p_ref[1]
    nd = _norm_from_deg(ddst_ref)
    h = jnp.maximum(agg * nd + b_ref[...], 0.0)
    ns = _norm_from_deg(dsrc_ref)
    o_ref[...] = h * ns


def _final_body(p_ref, ddst_ref, b_ref, w_ref, o_ref):
    agg = p_ref[0] + p_ref[1]
    nd = _norm_from_deg(ddst_ref)
    o_ref[...] = jnp.dot(agg * nd, w_ref[...],
                         preferred_element_type=jnp.float32) + b_ref[...]


def _entry(x, w, dsrc):
    dw = w.shape[1]
    return pl.pallas_call(
        _entry_body,
        grid=(NPAD // BLK,),
        in_specs=[
            pl.BlockSpec((BLK, D_IN), lambda i: (i, 0)),
            pl.BlockSpec((D_IN, dw), lambda i: (0, 0)),
            pl.BlockSpec((2, BLK, D_H), lambda i: (0, i, 0)),
        ],
        out_specs=pl.BlockSpec((BLK, dw), lambda i: (i, 0)),
        out_shape=jax.ShapeDtypeStruct((NPAD, dw), jnp.float32),
    )(x, w, dsrc)


def _mid(parts, ddst, dsrc, b2d, w):
    din, dw = w.shape
    return pl.pallas_call(
        _mid_body,
        grid=(NPAD // BLK,),
        in_specs=[
            pl.BlockSpec((2, BLK, din), lambda i: (0, i, 0)),
            pl.BlockSpec((2, BLK, D_H), lambda i: (0, i, 0)),
            pl.BlockSpec((2, BLK, D_H), lambda i: (0, i, 0)),
            pl.BlockSpec((1, din), lambda i: (0, 0)),
            pl.BlockSpec((din, dw), lambda i: (0, 0)),
        ],
        out_specs=pl.BlockSpec((BLK, dw), lambda i: (i, 0)),
        out_shape=jax.ShapeDtypeStruct((NPAD, dw), jnp.float32),
    )(parts, ddst, dsrc, b2d, w)


def _pre_final(parts, ddst, dsrc, b2d):
    return pl.pallas_call(
        _pre_final_body,
        grid=(NPAD // BLK,),
        in_specs=[
            pl.BlockSpec((2, BLK, D_H), lambda i: (0, i, 0)),
            pl.BlockSpec((2, BLK, D_H), lambda i: (0, i, 0)),
            pl.BlockSpec((2, BLK, D_H), lambda i: (0, i, 0)),
            pl.BlockSpec((1, D_H), lambda i: (0, 0)),
        ],
        out_specs=pl.BlockSpec((BLK, D_H), lambda i: (i, 0)),
        out_shape=jax.ShapeDtypeStruct((NPAD, D_H), jnp.float32),
    )(parts, ddst, dsrc, b2d)


def _final(parts, ddst, b2d, w):
    din, dw = w.shape
    return pl.pallas_call(
        _final_body,
        grid=(NPAD // BLK,),
        in_specs=[
            pl.BlockSpec((2, BLK, din), lambda i: (0, i, 0)),
            pl.BlockSpec((2, BLK, D_H), lambda i: (0, i, 0)),
            pl.BlockSpec((1, dw), lambda i: (0, 0)),
            pl.BlockSpec((din, dw), lambda i: (0, 0)),
        ],
        out_specs=pl.BlockSpec((BLK, dw), lambda i: (i, 0)),
        out_shape=jax.ShapeDtypeStruct((NPAD, dw), jnp.float32),
    )(parts, ddst, b2d, w)


_MESH = plsc.VectorSubcoreMesh(core_axis_name="c", subcore_axis_name="s")


def _make_agg(dw):
    @functools.partial(
        pl.kernel,
        mesh=_MESH,
        out_type=jax.ShapeDtypeStruct((2, NPAD, dw), jnp.float32),
        scratch_types=[
            pltpu.VMEM((HCH, CHUNK), jnp.int32),
            pltpu.VMEM((HCH, CHUNK), jnp.int32),
            pltpu.VMEM((CHUNK,), jnp.int32),
            pltpu.VMEM((CHUNK, dw), jnp.float32),
            pltpu.VMEM((CHUNK, dw), jnp.float32),
            pltpu.VMEM_SHARED((NPAD, dw), jnp.float32),
            pltpu.SemaphoreType.DMA,
            pltpu.SemaphoreType.DMA,
            pltpu.SemaphoreType.DMA,
            pltpu.SemaphoreType.DMA,
        ],
    )
    def agg(hs, srcp3, dstp3, zeros_hbm, dum_hbm, out, sidx, didx, dumidx,
            b0, b1, acc, g0, g1, s0, s1):
        bufs = (b0, b1)
        gsems = (g0, g1)
        ssems = (s0, s1)
        cid = lax.axis_index("c")
        sid = lax.axis_index("s")
        wid = sid * 2 + cid

        pltpu.sync_copy(dum_hbm, dumidx)
        for j in range(ZROWS // CHUNK):
            r0 = pl.multiple_of(sid * ZROWS + j * CHUNK, 8)
            pltpu.sync_copy(zeros_hbm, acc.at[pl.ds(r0, CHUNK)])
        plsc.subcore_barrier()

        for h in range(2):
            pltpu.sync_copy(srcp3.at[wid, pl.ds(h * HCH, HCH)], sidx)
            pltpu.sync_copy(dstp3.at[wid, pl.ds(h * HCH, HCH)], didx)
            pltpu.async_copy(hs.at[sidx.at[0]], bufs[0], gsems[0])
            pltpu.async_copy(bufs[1], acc.at[dumidx], ssems[1], add=True)

            def outer(o, _):
                for b in range(2):
                    g = o * 2 + b
                    pltpu.make_async_copy(hs.at[sidx.at[0]], bufs[b],
                                          gsems[b]).wait()
                    pltpu.async_copy(bufs[b], acc.at[didx.at[g]], ssems[b],
                                     add=True)
                    pltpu.make_async_copy(bufs[1 - b], acc.at[dumidx],
                                          ssems[1 - b]).wait()
                    gn = jnp.minimum(g + 1, HCH - 1)
                    pltpu.async_copy(hs.at[sidx.at[gn]], bufs[1 - b],
                                     gsems[1 - b])
                return 0

            lax.fori_loop(0, HCH // 2, outer, 0)
            pltpu.make_async_copy(hs.at[sidx.at[0]], bufs[0], gsems[0]).wait()
            pltpu.make_async_copy(bufs[1], acc.at[dumidx], ssems[1]).wait()
        plsc.subcore_barrier()

        for j in range(ZROWS // CHUNK):
            r0 = pl.multiple_of(sid * ZROWS + j * CHUNK, 8)
            pltpu.sync_copy(acc.at[pl.ds(r0, CHUNK)],
                            out.at[cid, pl.ds(r0, CHUNK)])

    return agg


_agg128 = _make_agg(D_H)


@functools.partial(
    pl.kernel,
    mesh=_MESH,
    out_type=(
        jax.ShapeDtypeStruct((2, NPAD, D_H), jnp.float32),
        jax.ShapeDtypeStruct((2, NPAD, D_H), jnp.float32),
    ),
    scratch_types=[
        pltpu.VMEM((HCH, CHUNK), jnp.int32),
        pltpu.VMEM((CHUNK,), jnp.int32),
        pltpu.VMEM((CHUNK, D_H), jnp.float32),
        pltpu.VMEM_SHARED((NPAD, D_H), jnp.float32),
        pltpu.SemaphoreType.DMA,
        pltpu.SemaphoreType.DMA,
    ],
)
def _deg(srcp3, dstp3, ones_hbm, zeros_hbm, dum_hbm, osrc, odst,
         idx, dumidx, ones_v, acc, s0, s1):
    sems = (s0, s1)
    cid = lax.axis_index("c")
    sid = lax.axis_index("s")
    wid = sid * 2 + cid

    pltpu.sync_copy(ones_hbm, ones_v)
    pltpu.sync_copy(dum_hbm, dumidx)

    def hist(ep3, out):
        for j in range(ZROWS // CHUNK):
            r0 = pl.multiple_of(sid * ZROWS + j * CHUNK, 8)
            pltpu.sync_copy(zeros_hbm, acc.at[pl.ds(r0, CHUNK)])
        plsc.subcore_barrier()

        for h in range(2):
            pltpu.sync_copy(ep3.at[wid, pl.ds(h * HCH, HCH)], idx)
            pltpu.async_copy(ones_v, acc.at[dumidx], sems[1], add=True)

            def outer(o, _):
                for b in range(2):
                    g = o * 2 + b
                    pltpu.async_copy(ones_v, acc.at[idx.at[g]], sems[b],
                                     add=True)
                    pltpu.make_async_copy(ones_v, acc.at[dumidx],
                                          sems[1 - b]).wait()
                return 0

            lax.fori_loop(0, HCH // 2, outer, 0)
            pltpu.make_async_copy(ones_v, acc.at[dumidx], sems[1]).wait()
        plsc.subcore_barrier()
        for j in range(ZROWS // CHUNK):
            r0 = pl.multiple_of(sid * ZROWS + j * CHUNK, 8)
            pltpu.sync_copy(acc.at[pl.ds(r0, CHUNK)],
                            out.at[cid, pl.ds(r0, CHUNK)])
        plsc.subcore_barrier()

    hist(srcp3, osrc)
    hist(dstp3, odst)


def kernel(features, edge_index, W0, b0, W1, b1, W2, b2):
    src = edge_index[0].astype(jnp.int32)
    dst = edge_index[1].astype(jnp.int32)
    pad = N + (jnp.arange(EPAD - E, dtype=jnp.int32) % (NPAD - N))
    srcp = jnp.concatenate([src, pad]).reshape(NW, NCH, CHUNK)
    dstp = jnp.concatenate([dst, pad]).reshape(NW, NCH, CHUNK)

    xpad = jnp.pad(features, ((0, NPAD - N), (0, 0)))
    w2p = jnp.pad(W2, ((0, 0), (0, D_OUT_PAD - D_OUT)))
    b0_2d = b0.reshape(1, D_H)
    b1_2d = b1.reshape(1, D_H)
    b2_2d = jnp.pad(b2, (0, D_OUT_PAD - D_OUT)).reshape(1, D_OUT_PAD)

    ones_tab = jnp.ones((CHUNK, D_H), jnp.float32)
    zeros_tab = jnp.zeros((CHUNK, D_H), jnp.float32)
    dum_tab = N + (jnp.arange(CHUNK, dtype=jnp.int32) % (NPAD - N))
    dsrc, ddst = _deg(srcp, dstp, ones_tab, zeros_tab, dum_tab)

    hs0 = _entry(xpad, W0, dsrc)
    p0 = _agg128(hs0, srcp, dstp, zeros_tab, dum_tab)
    hs1 = _mid(p0, ddst, dsrc, b0_2d, W1)
    p1 = _agg128(hs1, srcp, dstp, zeros_tab, dum_tab)
    hs2 = _pre_final(p1, ddst, dsrc, b1_2d)
    p2 = _agg128(hs2, srcp, dstp, zeros_tab, dum_tab)
    outp = _final(p2, ddst, b2_2d, w2p)
    return outp[:N, :D_OUT]


# speedup vs baseline: 1.0909x; 1.0909x over previous
import functools

import jax
import jax.numpy as jnp
from jax import lax
from jax.experimental import pallas as pl
from jax.experimental.pallas import tpu as pltpu
from jax.experimental.pallas import tpu_sc as plsc

N = 10000
E = 320000
D_IN = 128
D_H = 128
D_OUT = 40
D_OUT_PAD = 64

NPAD = 10240
BLK = 512
CHUNK = 128
NW = 32
NCH = 80
HCH = NCH // 2
EPT = NCH * CHUNK
EPAD = NW * EPT
ZROWS = NPAD // 16
NBUF = 2


def _mm_body(x_ref, w_ref, o_ref):
    o_ref[...] = jnp.dot(x_ref[...], w_ref[...],
                         preferred_element_type=jnp.float32)


def _norms_scale_body(dsrc_ref, ddst_ref, mm_ref, ns_ref, nd_ref, hs_ref):
    ds = dsrc_ref[0, :, :1] + dsrc_ref[1, :, :1]
    dd = ddst_ref[0, :, :1] + ddst_ref[1, :, :1]
    ns = jnp.where(ds > 0, lax.rsqrt(jnp.maximum(ds, 1.0)), 0.0)
    nd = jnp.where(dd > 0, lax.rsqrt(jnp.maximum(dd, 1.0)), 0.0)
    ns_ref[...] = jnp.broadcast_to(ns, (BLK, 8))
    nd_ref[...] = jnp.broadcast_to(nd, (BLK, 8))
    hs_ref[...] = mm_ref[...] * ns


def _mid_body(p_ref, nd_ref, ns_ref, b_ref, w_ref, o_ref):
    agg = p_ref[0] + p_ref[1]
    h = jnp.maximum(agg * nd_ref[:, :1] + b_ref[...], 0.0)
    o_ref[...] = jnp.dot(h, w_ref[...],
                         preferred_element_type=jnp.float32) * ns_ref[:, :1]


def _pre_final_body(p_ref, nd_ref, ns_ref, b_ref, o_ref):
    agg = p_ref[0] + p_ref[1]
    h = jnp.maximum(agg * nd_ref[:, :1] + b_ref[...], 0.0)
    o_ref[...] = h * ns_ref[:, :1]


def _final_body(p_ref, nd_ref, b_ref, w_ref, o_ref):
    agg = p_ref[0] + p_ref[1]
    o_ref[...] = jnp.dot(agg * nd_ref[:, :1], w_ref[...],
                         preferred_element_type=jnp.float32) + b_ref[...]


_NORM8 = pl.BlockSpec((BLK, 8), lambda i: (i, 0))


def _mm(x, w):
    dw = w.shape[1]
    return pl.pallas_call(
        _mm_body,
        grid=(NPAD // BLK,),
        in_specs=[
            pl.BlockSpec((BLK, D_IN), lambda i: (i, 0)),
            pl.BlockSpec((D_IN, dw), lambda i: (0, 0)),
        ],
        out_specs=pl.BlockSpec((BLK, dw), lambda i: (i, 0)),
        out_shape=jax.ShapeDtypeStruct((NPAD, dw), jnp.float32),
    )(x, w)


def _norms_scale(dsrc, ddst, mm0):
    return pl.pallas_call(
        _norms_scale_body,
        grid=(NPAD // BLK,),
        in_specs=[
            pl.BlockSpec((2, BLK, D_H), lambda i: (0, i, 0)),
            pl.BlockSpec((2, BLK, D_H), lambda i: (0, i, 0)),
            pl.BlockSpec((BLK, D_H), lambda i: (i, 0)),
        ],
        out_specs=[_NORM8, _NORM8, pl.BlockSpec((BLK, D_H), lambda i: (i, 0))],
        out_shape=[
            jax.ShapeDtypeStruct((NPAD, 8), jnp.float32),
            jax.ShapeDtypeStruct((NPAD, 8), jnp.float32),
            jax.ShapeDtypeStruct((NPAD, D_H), jnp.float32),
        ],
    )(dsrc, ddst, mm0)


def _mid(parts, nd8, ns8, b2d, w):
    din, dw = w.shape
    return pl.pallas_call(
        _mid_body,
        grid=(NPAD // BLK,),
        in_specs=[
            pl.BlockSpec((2, BLK, din), lambda i: (0, i, 0)),
            _NORM8,
            _NORM8,
            pl.BlockSpec((1, din), lambda i: (0, 0)),
            pl.BlockSpec((din, dw), lambda i: (0, 0)),
        ],
        out_specs=pl.BlockSpec((BLK, dw), lambda i: (i, 0)),
        out_shape=jax.ShapeDtypeStruct((NPAD, dw), jnp.float32),
    )(parts, nd8, ns8, b2d, w)


def _pre_final(parts, nd8, ns8, b2d):
    return pl.pallas_call(
        _pre_final_body,
        grid=(NPAD // BLK,),
        in_specs=[
            pl.BlockSpec((2, BLK, D_H), lambda i: (0, i, 0)),
            _NORM8,
            _NORM8,
            pl.BlockSpec((1, D_H), lambda i: (0, 0)),
        ],
        out_specs=pl.BlockSpec((BLK, D_H), lambda i: (i, 0)),
        out_shape=jax.ShapeDtypeStruct((NPAD, D_H), jnp.float32),
    )(parts, nd8, ns8, b2d)


def _final(parts, nd8, b2d, w):
    din, dw = w.shape
    return pl.pallas_call(
        _final_body,
        grid=(NPAD // BLK,),
        in_specs=[
            pl.BlockSpec((2, BLK, din), lambda i: (0, i, 0)),
            _NORM8,
            pl.BlockSpec((1, dw), lambda i: (0, 0)),
            pl.BlockSpec((din, dw), lambda i: (0, 0)),
        ],
        out_specs=pl.BlockSpec((BLK, dw), lambda i: (i, 0)),
        out_shape=jax.ShapeDtypeStruct((NPAD, dw), jnp.float32),
    )(parts, nd8, b2d, w)


_MESH = plsc.VectorSubcoreMesh(core_axis_name="c", subcore_axis_name="s")


def _make_agg(dw):
    @functools.partial(
        pl.kernel,
        mesh=_MESH,
        out_type=jax.ShapeDtypeStruct((2, NPAD, dw), jnp.float32),
        scratch_types=[
            pltpu.VMEM((HCH, CHUNK), jnp.int32),
            pltpu.VMEM((HCH, CHUNK), jnp.int32),
            pltpu.VMEM((CHUNK, dw), jnp.float32),
            pltpu.VMEM((CHUNK, dw), jnp.float32),
            pltpu.VMEM_SHARED((NPAD, dw), jnp.float32),
            pltpu.SemaphoreType.DMA,
            pltpu.SemaphoreType.DMA,
        ],
    )
    def agg(hs, srcp3, dstp3, zeros_hbm, dum_hbm, out, sidx, didx, b0, b1,
            acc, s0, s1):
        del dum_hbm
        bufs = (b0, b1)
        sems = (s0, s1)
        cid = lax.axis_index("c")
        sid = lax.axis_index("s")
        wid = sid * 2 + cid

        for j in range(ZROWS // CHUNK):
            r0 = pl.multiple_of(sid * ZROWS + j * CHUNK, 8)
            pltpu.sync_copy(zeros_hbm, acc.at[pl.ds(r0, CHUNK)])
        plsc.subcore_barrier()

        for h in range(2):
            pltpu.sync_copy(srcp3.at[wid, pl.ds(h * HCH, HCH)], sidx)
            pltpu.sync_copy(dstp3.at[wid, pl.ds(h * HCH, HCH)], didx)
            for b in range(NBUF):
                pltpu.make_async_copy(hs.at[sidx.at[b]], bufs[b],
                                      sems[b]).start()

            def outer(o, _):
                for b in range(NBUF):
                    g = o * NBUF + b
                    pltpu.make_async_copy(hs.at[sidx.at[0]], bufs[b],
                                          sems[b]).wait()
                    pltpu.sync_copy(bufs[b], acc.at[didx.at[g]], add=True)
                    gn = jnp.minimum(g + NBUF, HCH - 1)
                    pltpu.make_async_copy(hs.at[sidx.at[gn]], bufs[b],
                                          sems[b]).start()
                return 0

            lax.fori_loop(0, HCH // NBUF, outer, 0)
            for b in range(NBUF):
                pltpu.make_async_copy(hs.at[sidx.at[0]], bufs[b],
                                      sems[b]).wait()
        plsc.subcore_barrier()

        for j in range(ZROWS // CHUNK):
            r0 = pl.multiple_of(sid * ZROWS + j * CHUNK, 8)
            pltpu.sync_copy(acc.at[pl.ds(r0, CHUNK)],
                            out.at[cid, pl.ds(r0, CHUNK)])

    return agg


_agg128 = _make_agg(D_H)


@functools.partial(
    pl.kernel,
    mesh=_MESH,
    out_type=(
        jax.ShapeDtypeStruct((2, NPAD, D_H), jnp.float32),
        jax.ShapeDtypeStruct((2, NPAD, D_H), jnp.float32),
    ),
    scratch_types=[
        pltpu.VMEM((HCH, CHUNK), jnp.int32),
        pltpu.VMEM((CHUNK, D_H), jnp.float32),
        pltpu.VMEM_SHARED((NPAD, D_H), jnp.float32),
    ],
)
def _deg(srcp3, dstp3, ones_hbm, zeros_hbm, dum_hbm, osrc, odst,
         idx, ones_v, acc):
    del dum_hbm
    cid = lax.axis_index("c")
    sid = lax.axis_index("s")
    wid = sid * 2 + cid

    pltpu.sync_copy(ones_hbm, ones_v)

    def hist(ep3, out):
        for j in range(ZROWS // CHUNK):
            r0 = pl.multiple_of(sid * ZROWS + j * CHUNK, 8)
            pltpu.sync_copy(zeros_hbm, acc.at[pl.ds(r0, CHUNK)])
        plsc.subcore_barrier()

        for h in range(2):
            pltpu.sync_copy(ep3.at[wid, pl.ds(h * HCH, HCH)], idx)

            def body(g, _):
                pltpu.sync_copy(ones_v, acc.at[idx.at[g]], add=True)
                return 0

            lax.fori_loop(0, HCH, body, 0)
        plsc.subcore_barrier()
        for j in range(ZROWS // CHUNK):
            r0 = pl.multiple_of(sid * ZROWS + j * CHUNK, 8)
            pltpu.sync_copy(acc.at[pl.ds(r0, CHUNK)],
                            out.at[cid, pl.ds(r0, CHUNK)])
        plsc.subcore_barrier()

    hist(srcp3, osrc)
    hist(dstp3, odst)


def kernel(features, edge_index, W0, b0, W1, b1, W2, b2):
    src = edge_index[0].astype(jnp.int32)
    dst = edge_index[1].astype(jnp.int32)
    pad = N + (jnp.arange(EPAD - E, dtype=jnp.int32) % (NPAD - N))
    srcp = jnp.concatenate([src, pad]).reshape(NW, NCH, CHUNK)
    dstp = jnp.concatenate([dst, pad]).reshape(NW, NCH, CHUNK)

    xpad = jnp.pad(features, ((0, NPAD - N), (0, 0)))
    w2p = jnp.pad(W2, ((0, 0), (0, D_OUT_PAD - D_OUT)))
    b0_2d = b0.reshape(1, D_H)
    b1_2d = b1.reshape(1, D_H)
    b2_2d = jnp.pad(b2, (0, D_OUT_PAD - D_OUT)).reshape(1, D_OUT_PAD)

    ones_tab = jnp.ones((CHUNK, D_H), jnp.float32)
    zeros_tab = jnp.zeros((CHUNK, D_H), jnp.float32)
    dum_tab = N + (jnp.arange(CHUNK, dtype=jnp.int32) % (NPAD - N))
    mm0 = _mm(xpad, W0)
    dsrc, ddst = _deg(srcp, dstp, ones_tab, zeros_tab, dum_tab)
    ns8, nd8, hs0 = _norms_scale(dsrc, ddst, mm0)

    p0 = _agg128(hs0, srcp, dstp, zeros_tab, dum_tab)
    hs1 = _mid(p0, nd8, ns8, b0_2d, W1)
    p1 = _agg128(hs1, srcp, dstp, zeros_tab, dum_tab)
    hs2 = _pre_final(p1, nd8, ns8, b1_2d)
    p2 = _agg128(hs2, srcp, dstp, zeros_tab, dum_tab)
    outp = _final(p2, nd8, b2_2d, w2p)
    return outp[:N, :D_OUT]


# speedup vs baseline: 1.1856x; 1.0868x over previous
import functools

import jax
import jax.numpy as jnp
from jax import lax
from jax.experimental import pallas as pl
from jax.experimental.pallas import tpu as pltpu
from jax.experimental.pallas import tpu_sc as plsc

N = 10000
E = 320000
D_IN = 128
D_H = 128
D_OUT = 40
D_OUT_PAD = 64

NPAD = 10240
BLK = 512
CHUNK = 128
NW = 32
NCH = 80
HCH = NCH // 2
EPT = NCH * CHUNK
EPAD = NW * EPT
ZROWS = NPAD // 16
NBUF = 2


def _mm_body(x_ref, w_ref, o_ref):
    o_ref[...] = jnp.dot(x_ref[...], w_ref[...],
                         preferred_element_type=jnp.float32)


def _norms_scale_body(dsrc_ref, ddst_ref, mm_ref, ns_ref, nd_ref, hs_ref):
    ds = dsrc_ref[0, :, :1] + dsrc_ref[1, :, :1]
    dd = ddst_ref[0, :, :1] + ddst_ref[1, :, :1]
    ns = jnp.where(ds > 0, lax.rsqrt(jnp.maximum(ds, 1.0)), 0.0)
    nd = jnp.where(dd > 0, lax.rsqrt(jnp.maximum(dd, 1.0)), 0.0)
    ns_ref[...] = jnp.broadcast_to(ns, (BLK, 8))
    nd_ref[...] = jnp.broadcast_to(nd, (BLK, 8))
    hs_ref[...] = mm_ref[...] * ns


def _mid_body(p_ref, nd_ref, ns_ref, b_ref, w_ref, o_ref):
    agg = p_ref[0] + p_ref[1]
    h = jnp.maximum(agg * nd_ref[:, :1] + b_ref[...], 0.0)
    o_ref[...] = jnp.dot(h, w_ref[...],
                         preferred_element_type=jnp.float32) * ns_ref[:, :1]


def _pre_final_body(p_ref, nd_ref, ns_ref, b_ref, o_ref):
    agg = p_ref[0] + p_ref[1]
    h = jnp.maximum(agg * nd_ref[:, :1] + b_ref[...], 0.0)
    o_ref[...] = h * ns_ref[:, :1]


def _final_body(p_ref, nd_ref, b_ref, w_ref, o_ref):
    agg = p_ref[0] + p_ref[1]
    o_ref[...] = jnp.dot(agg * nd_ref[:, :1], w_ref[...],
                         preferred_element_type=jnp.float32) + b_ref[...]


_NORM8 = pl.BlockSpec((BLK, 8), lambda i: (i, 0))


def _mm(x, w):
    dw = w.shape[1]
    return pl.pallas_call(
        _mm_body,
        grid=(NPAD // BLK,),
        in_specs=[
            pl.BlockSpec((BLK, D_IN), lambda i: (i, 0)),
            pl.BlockSpec((D_IN, dw), lambda i: (0, 0)),
        ],
        out_specs=pl.BlockSpec((BLK, dw), lambda i: (i, 0)),
        out_shape=jax.ShapeDtypeStruct((NPAD, dw), jnp.float32),
    )(x, w)


def _norms_scale(dsrc, ddst, mm0):
    return pl.pallas_call(
        _norms_scale_body,
        grid=(NPAD // BLK,),
        in_specs=[
            pl.BlockSpec((2, BLK, D_H), lambda i: (0, i, 0)),
            pl.BlockSpec((2, BLK, D_H), lambda i: (0, i, 0)),
            pl.BlockSpec((BLK, D_H), lambda i: (i, 0)),
        ],
        out_specs=[_NORM8, _NORM8, pl.BlockSpec((BLK, D_H), lambda i: (i, 0))],
        out_shape=[
            jax.ShapeDtypeStruct((NPAD, 8), jnp.float32),
            jax.ShapeDtypeStruct((NPAD, 8), jnp.float32),
            jax.ShapeDtypeStruct((NPAD, D_H), jnp.float32),
        ],
    )(dsrc, ddst, mm0)


def _mid(parts, nd8, ns8, b2d, w):
    din, dw = w.shape
    return pl.pallas_call(
        _mid_body,
        grid=(NPAD // BLK,),
        in_specs=[
            pl.BlockSpec((2, BLK, din), lambda i: (0, i, 0)),
            _NORM8,
            _NORM8,
            pl.BlockSpec((1, din), lambda i: (0, 0)),
            pl.BlockSpec((din, dw), lambda i: (0, 0)),
        ],
        out_specs=pl.BlockSpec((BLK, dw), lambda i: (i, 0)),
        out_shape=jax.ShapeDtypeStruct((NPAD, dw), jnp.float32),
    )(parts, nd8, ns8, b2d, w)


def _pre_final(parts, nd8, ns8, b2d):
    return pl.pallas_call(
        _pre_final_body,
        grid=(NPAD // BLK,),
        in_specs=[
            pl.BlockSpec((2, BLK, D_H), lambda i: (0, i, 0)),
            _NORM8,
            _NORM8,
            pl.BlockSpec((1, D_H), lambda i: (0, 0)),
        ],
        out_specs=pl.BlockSpec((BLK, D_H), lambda i: (i, 0)),
        out_shape=jax.ShapeDtypeStruct((NPAD, D_H), jnp.float32),
    )(parts, nd8, ns8, b2d)


def _final(parts, nd8, b2d, w):
    din, dw = w.shape
    return pl.pallas_call(
        _final_body,
        grid=(NPAD // BLK,),
        in_specs=[
            pl.BlockSpec((2, BLK, din), lambda i: (0, i, 0)),
            _NORM8,
            pl.BlockSpec((1, dw), lambda i: (0, 0)),
            pl.BlockSpec((din, dw), lambda i: (0, 0)),
        ],
        out_specs=pl.BlockSpec((BLK, dw), lambda i: (i, 0)),
        out_shape=jax.ShapeDtypeStruct((NPAD, dw), jnp.float32),
    )(parts, nd8, b2d, w)


_MESH = plsc.VectorSubcoreMesh(core_axis_name="c", subcore_axis_name="s")


def _make_agg(dw):
    @functools.partial(
        pl.kernel,
        mesh=_MESH,
        out_type=jax.ShapeDtypeStruct((2, NPAD, dw), jnp.float32),
        scratch_types=[
            pltpu.VMEM((HCH, CHUNK), jnp.int32),
            pltpu.VMEM((HCH, CHUNK), jnp.int32),
            pltpu.VMEM((CHUNK, dw), jnp.float32),
            pltpu.VMEM((CHUNK, dw), jnp.float32),
            pltpu.VMEM_SHARED((NPAD, dw), jnp.float32),
            pltpu.SemaphoreType.DMA,
            pltpu.SemaphoreType.DMA,
        ],
    )
    def agg(hs, srcp3, dstp3, zeros_hbm, dum_hbm, out, sidx, didx, b0, b1,
            acc, s0, s1):
        del dum_hbm
        bufs = (b0, b1)
        sems = (s0, s1)
        cid = lax.axis_index("c")
        sid = lax.axis_index("s")
        wid = sid * 2 + cid

        zbase = pl.multiple_of(sid * ZROWS, 8)
        pltpu.sync_copy(zeros_hbm, acc.at[pl.ds(zbase, ZROWS)])
        plsc.subcore_barrier()

        for h in range(2):
            pltpu.sync_copy(srcp3.at[wid, pl.ds(h * HCH, HCH)], sidx)
            pltpu.sync_copy(dstp3.at[wid, pl.ds(h * HCH, HCH)], didx)
            for b in range(NBUF):
                pltpu.make_async_copy(hs.at[sidx.at[b]], bufs[b],
                                      sems[b]).start()

            def outer(o, _):
                for b in range(NBUF):
                    g = o * NBUF + b
                    pltpu.make_async_copy(hs.at[sidx.at[0]], bufs[b],
                                          sems[b]).wait()
                    pltpu.sync_copy(bufs[b], acc.at[didx.at[g]], add=True)
                    gn = jnp.minimum(g + NBUF, HCH - 1)
                    pltpu.make_async_copy(hs.at[sidx.at[gn]], bufs[b],
                                          sems[b]).start()
                return 0

            lax.fori_loop(0, HCH // NBUF, outer, 0)
            for b in range(NBUF):
                pltpu.make_async_copy(hs.at[sidx.at[0]], bufs[b],
                                      sems[b]).wait()
        plsc.subcore_barrier()

        pltpu.sync_copy(acc.at[pl.ds(zbase, ZROWS)],
                        out.at[cid, pl.ds(zbase, ZROWS)])

    return agg


_agg128 = _make_agg(D_H)


@functools.partial(
    pl.kernel,
    mesh=_MESH,
    out_type=(
        jax.ShapeDtypeStruct((2, NPAD, D_H), jnp.float32),
        jax.ShapeDtypeStruct((2, NPAD, D_H), jnp.float32),
    ),
    scratch_types=[
        pltpu.VMEM((HCH, CHUNK), jnp.int32),
        pltpu.VMEM((CHUNK, D_H), jnp.float32),
        pltpu.VMEM_SHARED((NPAD, D_H), jnp.float32),
    ],
)
def _deg(srcp3, dstp3, ones_hbm, zeros_hbm, dum_hbm, osrc, odst,
         idx, ones_v, acc):
    del dum_hbm
    cid = lax.axis_index("c")
    sid = lax.axis_index("s")
    wid = sid * 2 + cid

    pltpu.sync_copy(ones_hbm, ones_v)

    def hist(ep3, out):
        zbase = pl.multiple_of(sid * ZROWS, 8)
        pltpu.sync_copy(zeros_hbm, acc.at[pl.ds(zbase, ZROWS)])
        plsc.subcore_barrier()

        for h in range(2):
            pltpu.sync_copy(ep3.at[wid, pl.ds(h * HCH, HCH)], idx)

            def body(g, _):
                pltpu.sync_copy(ones_v, acc.at[idx.at[g]], add=True)
                return 0

            lax.fori_loop(0, HCH, body, 0)
        plsc.subcore_barrier()
        pltpu.sync_copy(acc.at[pl.ds(zbase, ZROWS)],
                        out.at[cid, pl.ds(zbase, ZROWS)])
        plsc.subcore_barrier()

    hist(srcp3, osrc)
    hist(dstp3, odst)


def kernel(features, edge_index, W0, b0, W1, b1, W2, b2):
    src = edge_index[0].astype(jnp.int32)
    dst = edge_index[1].astype(jnp.int32)
    pad = N + (jnp.arange(EPAD - E, dtype=jnp.int32) % (NPAD - N))
    srcp = jnp.concatenate([src, pad]).reshape(NW, NCH, CHUNK)
    dstp = jnp.concatenate([dst, pad]).reshape(NW, NCH, CHUNK)

    xpad = jnp.pad(features, ((0, NPAD - N), (0, 0)))
    w2p = jnp.pad(W2, ((0, 0), (0, D_OUT_PAD - D_OUT)))
    b0_2d = b0.reshape(1, D_H)
    b1_2d = b1.reshape(1, D_H)
    b2_2d = jnp.pad(b2, (0, D_OUT_PAD - D_OUT)).reshape(1, D_OUT_PAD)

    ones_tab = jnp.ones((CHUNK, D_H), jnp.float32)
    zeros_tab = jnp.zeros((ZROWS, D_H), jnp.float32)
    dum_tab = N + (jnp.arange(CHUNK, dtype=jnp.int32) % (NPAD - N))
    mm0 = _mm(xpad, W0)
    dsrc, ddst = _deg(srcp, dstp, ones_tab, zeros_tab, dum_tab)
    ns8, nd8, hs0 = _norms_scale(dsrc, ddst, mm0)

    p0 = _agg128(hs0, srcp, dstp, zeros_tab, dum_tab)
    hs1 = _mid(p0, nd8, ns8, b0_2d, W1)
    p1 = _agg128(hs1, srcp, dstp, zeros_tab, dum_tab)
    hs2 = _pre_final(p1, nd8, ns8, b1_2d)
    p2 = _agg128(hs2, srcp, dstp, zeros_tab, dum_tab)
    outp = _final(p2, nd8, b2_2d, w2p)
    return outp[:N, :D_OUT]
